# baseline (speedup 1.0000x reference)
import jax
import jax.numpy as jnp
from jax import lax
from jax.experimental import pallas as pl
from jax.experimental.pallas import tpu as pltpu
from jax.experimental.pallas import tpu_sc as plsc

NC = 2
NS = 16
NW = NC * NS
CHUNK = 128
LANES = 16

VOCAB = 100000
EMB = 64
BATCH = 16384

TBW = 16384
NBLK = (VOCAB + TBW - 1) // TBW
VP = NBLK * TBW

TSH = TBW.bit_length() - 1

B_PER_W = BATCH // NW
CHUNKS_PER_W = B_PER_W // CHUNK


def _pack_body(d_ref, p_ref, eye_ref, od_ref, op_ref):
    eye = eye_ref[...]
    dims = (((0,), (0,)), ((), ()))
    x = d_ref[...]
    od_ref[:, 0:EMB] = lax.dot_general(
        x[:, 0:TBW // 2], eye, dims, preferred_element_type=jnp.float32)
    od_ref[:, EMB:128] = lax.dot_general(
        x[:, TBW // 2:TBW], eye, dims, preferred_element_type=jnp.float32)
    y = p_ref[...]
    op_ref[:, 0:EMB] = lax.dot_general(
        y[:, 0:TBW // 2], eye, dims, preferred_element_type=jnp.float32)
    op_ref[:, EMB:128] = lax.dot_general(
        y[:, TBW // 2:TBW], eye, dims, preferred_element_type=jnp.float32)


def _pack(dT, pT, eye):
    return pl.pallas_call(
        _pack_body,
        grid=(NBLK,),
        in_specs=[
            pl.BlockSpec((EMB, TBW), lambda i: (0, i)),
            pl.BlockSpec((EMB, TBW), lambda i: (0, i)),
            pl.BlockSpec((EMB, EMB), lambda i: (0, 0)),
        ],
        out_specs=[
            pl.BlockSpec((TBW // 2, 128), lambda i: (i, 0)),
            pl.BlockSpec((TBW // 2, 128), lambda i: (i, 0)),
        ],
        out_shape=[
            jax.ShapeDtypeStruct((VP // 2, 128), jnp.float32),
            jax.ShapeDtypeStruct((VP // 2, 128), jnp.float32),
        ],
    )(dT, pT, eye)


def _remap(idx_ref):
    for j in range(CHUNKS_PER_W):
        for k in range(CHUNK // LANES):
            v = idx_ref[j, pl.ds(k * LANES, LANES)]
            jj = (((v >> TSH) << TSH) + ((v & (TBW // 2 - 1)) << 1)
                  + ((v >> (TSH - 1)) & 1))
            idx_ref[j, pl.ds(k * LANES, LANES)] = jj


def _gather_body(d_lin, p_lin, d_ids, p_ids, d_out, p_out,
                 idx_d, idx_p, rows_d, rows_p, sem):
    wid = lax.axis_index("s") * NC + lax.axis_index("c")
    rbase = wid * CHUNKS_PER_W
    pltpu.sync_copy(d_ids.at[pl.ds(rbase, CHUNKS_PER_W)], idx_d)
    pltpu.sync_copy(p_ids.at[pl.ds(rbase, CHUNKS_PER_W)], idx_p)
    _remap(idx_d)
    _remap(idx_p)
    copies = []
    for j in range(CHUNKS_PER_W):
        copies.append(pltpu.async_copy(
            d_lin.at[idx_d.at[j]], rows_d.at[pl.ds(j * CHUNK, CHUNK)], sem))
        copies.append(pltpu.async_copy(
            p_lin.at[idx_p.at[j]], rows_p.at[pl.ds(j * CHUNK, CHUNK)], sem))
    for c in copies:
        c.wait()
    base = wid * B_PER_W
    pltpu.sync_copy(rows_d, d_out.at[pl.ds(base, B_PER_W)])
    pltpu.sync_copy(rows_p, p_out.at[pl.ds(base, B_PER_W)])


_gather = pl.kernel(
    _gather_body,
    out_type=(
        jax.ShapeDtypeStruct((BATCH, EMB), jnp.float32),
        jax.ShapeDtypeStruct((BATCH, EMB), jnp.float32),
    ),
    mesh=plsc.VectorSubcoreMesh(
        core_axis_name="c", subcore_axis_name="s",
        num_cores=NC, num_subcores=NS),
    scratch_types=[
        pltpu.VMEM((CHUNKS_PER_W, CHUNK), jnp.int32),
        pltpu.VMEM((CHUNKS_PER_W, CHUNK), jnp.int32),
        pltpu.VMEM((B_PER_W, EMB), jnp.float32),
        pltpu.VMEM((B_PER_W, EMB), jnp.float32),
        pltpu.SemaphoreType.DMA,
    ],
    compiler_params=pltpu.CompilerParams(use_tc_tiling_on_sc=False),
)


def _mlp_body(d_ref, p_ref, w1a_ref, w1b_ref, b1_ref, w2_ref, b2_ref, o_ref):
    h = jnp.dot(d_ref[...], w1a_ref[...], preferred_element_type=jnp.float32)
    h = h + jnp.dot(p_ref[...], w1b_ref[...], preferred_element_type=jnp.float32)
    h = jnp.maximum(h + b1_ref[...], 0.0)
    o = jnp.dot(h, w2_ref[...], preferred_element_type=jnp.float32) + b2_ref[...]
    o_ref[...] = 1.0 / (1.0 + jnp.exp(-o))


BB = 2048


def _mlp(d2, p2, w1a_bd, w1b_bd, b1_bd, w2_bd, b2_bd):
    return pl.pallas_call(
        _mlp_body,
        grid=(BATCH // 2 // BB,),
        in_specs=[
            pl.BlockSpec((BB, 128), lambda i: (i, 0)),
            pl.BlockSpec((BB, 128), lambda i: (i, 0)),
            pl.BlockSpec((128, 2 * 32), lambda i: (0, 0)),
            pl.BlockSpec((128, 2 * 32), lambda i: (0, 0)),
            pl.BlockSpec((1, 2 * 32), lambda i: (0, 0)),
            pl.BlockSpec((2 * 32, 8), lambda i: (0, 0)),
            pl.BlockSpec((1, 8), lambda i: (0, 0)),
        ],
        out_specs=pl.BlockSpec((BB, 8), lambda i: (i, 0)),
        out_shape=jax.ShapeDtypeStruct((BATCH // 2, 8), jnp.float32),
    )(d2, p2, w1a_bd, w1b_bd, b1_bd, w2_bd, b2_bd)


def _block_diag(w):
    r, c = w.shape
    z = jnp.zeros((r, c), dtype=w.dtype)
    return jnp.concatenate(
        [jnp.concatenate([w, z], axis=1),
         jnp.concatenate([z, w], axis=1)], axis=0)


@jax.jit
def kernel(depth_ids, purpose_ids, depth_table, purpose_table, W1, b1, W2, b2):
    d_ids = depth_ids.astype(jnp.int32).reshape(NW * CHUNKS_PER_W, CHUNK)
    p_ids = purpose_ids.astype(jnp.int32).reshape(NW * CHUNKS_PER_W, CHUNK)
    eye = jnp.eye(EMB, dtype=jnp.float32)
    lin_d, lin_p = _pack(depth_table.T, purpose_table.T, eye)
    d_emb, p_emb = _gather(lin_d.reshape(VP, EMB), lin_p.reshape(VP, EMB),
                           d_ids, p_ids)
    d2 = d_emb.reshape(BATCH // 2, 128)
    p2 = p_emb.reshape(BATCH // 2, 128)
    w1a_bd = _block_diag(W1[:EMB])
    w1b_bd = _block_diag(W1[EMB:])
    b1_bd = jnp.concatenate([b1, b1]).reshape(1, 2 * 32)
    w2_bd = _block_diag(W2)
    b2_bd = jnp.concatenate([b2, b2]).reshape(1, 8)
    out2 = _mlp(d2, p2, w1a_bd, w1b_bd, b1_bd, w2_bd, b2_bd)
    return out2.reshape(BATCH, 4)


# speedup vs baseline: 1.1116x; 1.1116x over previous
import jax
import jax.numpy as jnp
from jax import lax
from jax.experimental import pallas as pl
from jax.experimental.pallas import tpu as pltpu
from jax.experimental.pallas import tpu_sc as plsc

NC = 2
NS = 16
NW = NC * NS
CHUNK = 128
LANES = 16

VOCAB = 100000
EMB = 64
BATCH = 16384

TBW = 8192
NBLK = (VOCAB + TBW - 1) // TBW
VP = NBLK * TBW

TSH = TBW.bit_length() - 1

B_PER_W = BATCH // NW
CHUNKS_PER_W = B_PER_W // CHUNK


def _pack_body(d_ref, p_ref, eye_ref, od_ref, op_ref):
    eye = eye_ref[...]
    dims = (((0,), (0,)), ((), ()))
    x = d_ref[...]
    od_ref[:, 0:EMB] = lax.dot_general(
        x[:, 0:TBW // 2], eye, dims, preferred_element_type=jnp.float32)
    od_ref[:, EMB:128] = lax.dot_general(
        x[:, TBW // 2:TBW], eye, dims, preferred_element_type=jnp.float32)
    y = p_ref[...]
    op_ref[:, 0:EMB] = lax.dot_general(
        y[:, 0:TBW // 2], eye, dims, preferred_element_type=jnp.float32)
    op_ref[:, EMB:128] = lax.dot_general(
        y[:, TBW // 2:TBW], eye, dims, preferred_element_type=jnp.float32)


def _pack(dT, pT, eye):
    return pl.pallas_call(
        _pack_body,
        grid=(NBLK,),
        in_specs=[
            pl.BlockSpec((EMB, TBW), lambda i: (0, i)),
            pl.BlockSpec((EMB, TBW), lambda i: (0, i)),
            pl.BlockSpec((EMB, EMB), lambda i: (0, 0)),
        ],
        out_specs=[
            pl.BlockSpec((TBW // 2, 128), lambda i: (i, 0)),
            pl.BlockSpec((TBW // 2, 128), lambda i: (i, 0)),
        ],
        out_shape=[
            jax.ShapeDtypeStruct((VP // 2, 128), jnp.float32),
            jax.ShapeDtypeStruct((VP // 2, 128), jnp.float32),
        ],
    )(dT, pT, eye)


def _remap(idx_ref):
    for j in range(CHUNKS_PER_W):
        for k in range(CHUNK // LANES):
            v = idx_ref[j, pl.ds(k * LANES, LANES)]
            jj = (((v >> TSH) << TSH) + ((v & (TBW // 2 - 1)) << 1)
                  + ((v >> (TSH - 1)) & 1))
            idx_ref[j, pl.ds(k * LANES, LANES)] = jj


def _gather_body(d_lin, p_lin, d_ids, p_ids, d_out, p_out,
                 idx_d, idx_p, rows_d, rows_p, sem):
    wid = lax.axis_index("s") * NC + lax.axis_index("c")
    rbase = wid * CHUNKS_PER_W
    pltpu.sync_copy(d_ids.at[pl.ds(rbase, CHUNKS_PER_W)], idx_d)
    pltpu.sync_copy(p_ids.at[pl.ds(rbase, CHUNKS_PER_W)], idx_p)
    _remap(idx_d)
    _remap(idx_p)
    copies = []
    for j in range(CHUNKS_PER_W):
        copies.append(pltpu.async_copy(
            d_lin.at[idx_d.at[j]], rows_d.at[pl.ds(j * CHUNK, CHUNK)], sem))
        copies.append(pltpu.async_copy(
            p_lin.at[idx_p.at[j]], rows_p.at[pl.ds(j * CHUNK, CHUNK)], sem))
    for c in copies:
        c.wait()
    half = wid // (NW // 2)
    base = (wid % (NW // 2)) * B_PER_W
    pltpu.sync_copy(rows_d,
                    d_out.at[pl.ds(base, B_PER_W), pl.ds(half * EMB, EMB)])
    pltpu.sync_copy(rows_p,
                    p_out.at[pl.ds(base, B_PER_W), pl.ds(half * EMB, EMB)])


_gather = pl.kernel(
    _gather_body,
    out_type=(
        jax.ShapeDtypeStruct((BATCH // 2, 2 * EMB), jnp.float32),
        jax.ShapeDtypeStruct((BATCH // 2, 2 * EMB), jnp.float32),
    ),
    mesh=plsc.VectorSubcoreMesh(
        core_axis_name="c", subcore_axis_name="s",
        num_cores=NC, num_subcores=NS),
    scratch_types=[
        pltpu.VMEM((CHUNKS_PER_W, CHUNK), jnp.int32),
        pltpu.VMEM((CHUNKS_PER_W, CHUNK), jnp.int32),
        pltpu.VMEM((B_PER_W, EMB), jnp.float32),
        pltpu.VMEM((B_PER_W, EMB), jnp.float32),
        pltpu.SemaphoreType.DMA,
    ],
    compiler_params=pltpu.CompilerParams(use_tc_tiling_on_sc=False),
)


def _mlp_body(d_ref, p_ref, w1a_ref, w1b_ref, b1_ref, w2_ref, b2_ref,
              otop_ref, obot_ref):
    h = jnp.dot(d_ref[...], w1a_ref[...], preferred_element_type=jnp.float32)
    h = h + jnp.dot(p_ref[...], w1b_ref[...], preferred_element_type=jnp.float32)
    h = jnp.maximum(h + b1_ref[...], 0.0)
    o = jnp.dot(h, w2_ref[...], preferred_element_type=jnp.float32) + b2_ref[...]
    o = 1.0 / (1.0 + jnp.exp(-o))
    otop_ref[...] = o[:, 0:4]
    obot_ref[...] = o[:, 4:8]


BB = 2048


def _mlp(d2, p2, w1a_bd, w1b_bd, b1_bd, w2_bd, b2_bd):
    return pl.pallas_call(
        _mlp_body,
        grid=(BATCH // 2 // BB,),
        in_specs=[
            pl.BlockSpec((BB, 128), lambda i: (i, 0)),
            pl.BlockSpec((BB, 128), lambda i: (i, 0)),
            pl.BlockSpec((128, 2 * 32), lambda i: (0, 0)),
            pl.BlockSpec((128, 2 * 32), lambda i: (0, 0)),
            pl.BlockSpec((1, 2 * 32), lambda i: (0, 0)),
            pl.BlockSpec((2 * 32, 8), lambda i: (0, 0)),
            pl.BlockSpec((1, 8), lambda i: (0, 0)),
        ],
        out_specs=[pl.BlockSpec((BB, 4), lambda i: (i, 0)),
                   pl.BlockSpec((BB, 4), lambda i: (i, 0))],
        out_shape=[jax.ShapeDtypeStruct((BATCH // 2, 4), jnp.float32),
                   jax.ShapeDtypeStruct((BATCH // 2, 4), jnp.float32)],
    )(d2, p2, w1a_bd, w1b_bd, b1_bd, w2_bd, b2_bd)


def _block_diag(w):
    r, c = w.shape
    z = jnp.zeros((r, c), dtype=w.dtype)
    return jnp.concatenate(
        [jnp.concatenate([w, z], axis=1),
         jnp.concatenate([z, w], axis=1)], axis=0)


@jax.jit
def kernel(depth_ids, purpose_ids, depth_table, purpose_table, W1, b1, W2, b2):
    d_ids = depth_ids.astype(jnp.int32).reshape(NW * CHUNKS_PER_W, CHUNK)
    p_ids = purpose_ids.astype(jnp.int32).reshape(NW * CHUNKS_PER_W, CHUNK)
    eye = jnp.eye(EMB, dtype=jnp.float32)
    lin_d, lin_p = _pack(depth_table.T, purpose_table.T, eye)
    d2, p2 = _gather(lin_d.reshape(VP, EMB), lin_p.reshape(VP, EMB),
                     d_ids, p_ids)
    w1a_bd = _block_diag(W1[:EMB])
    w1b_bd = _block_diag(W1[EMB:])
    b1_bd = jnp.concatenate([b1, b1]).reshape(1, 2 * 32)
    w2_bd = _block_diag(W2)
    b2_bd = jnp.concatenate([b2, b2]).reshape(1, 8)
    o_top, o_bot = _mlp(d2, p2, w1a_bd, w1b_bd, b1_bd, w2_bd, b2_bd)
    return jnp.concatenate([o_top, o_bot], axis=0)


# speedup vs baseline: 1.1326x; 1.0189x over previous
import jax
import jax.numpy as jnp
from jax import lax
from jax.experimental import pallas as pl
from jax.experimental.pallas import tpu as pltpu
from jax.experimental.pallas import tpu_sc as plsc

NC = 2
NS = 16
NW = NC * NS
CHUNK = 128
LANES = 16

VOCAB = 100000
EMB = 64
BATCH = 16384

TBW = 8192
NBLK = (VOCAB + TBW - 1) // TBW
VP = NBLK * TBW

TSH = TBW.bit_length() - 1

B_PER_W = BATCH // NW
CHUNKS_PER_W = B_PER_W // CHUNK


def _pack_body(d_ref, p_ref, eye_ref, od_ref, op_ref):
    eye = eye_ref[...]
    dims = (((0,), (0,)), ((), ()))
    x = d_ref[...]
    od_ref[:, 0:EMB] = lax.dot_general(
        x[:, 0:TBW // 2], eye, dims, preferred_element_type=jnp.float32)
    od_ref[:, EMB:128] = lax.dot_general(
        x[:, TBW // 2:TBW], eye, dims, preferred_element_type=jnp.float32)
    y = p_ref[...]
    op_ref[:, 0:EMB] = lax.dot_general(
        y[:, 0:TBW // 2], eye, dims, preferred_element_type=jnp.float32)
    op_ref[:, EMB:128] = lax.dot_general(
        y[:, TBW // 2:TBW], eye, dims, preferred_element_type=jnp.float32)


def _pack(dT, pT, eye):
    return pl.pallas_call(
        _pack_body,
        grid=(NBLK,),
        in_specs=[
            pl.BlockSpec((EMB, TBW), lambda i: (0, i)),
            pl.BlockSpec((EMB, TBW), lambda i: (0, i)),
            pl.BlockSpec((EMB, EMB), lambda i: (0, 0)),
        ],
        out_specs=[
            pl.BlockSpec((TBW // 2, 128), lambda i: (i, 0)),
            pl.BlockSpec((TBW // 2, 128), lambda i: (i, 0)),
        ],
        out_shape=[
            jax.ShapeDtypeStruct((VP // 2, 128), jnp.float32),
            jax.ShapeDtypeStruct((VP // 2, 128), jnp.float32),
        ],
    )(dT, pT, eye)


def _remap(idx_ref):
    for j in range(CHUNKS_PER_W):
        for k in range(CHUNK // LANES):
            v = idx_ref[j, pl.ds(k * LANES, LANES)]
            jj = (((v >> TSH) << TSH) + ((v & (TBW // 2 - 1)) << 1)
                  + ((v >> (TSH - 1)) & 1))
            idx_ref[j, pl.ds(k * LANES, LANES)] = jj


def _gather_body(d_lin, p_lin, d_ids, p_ids, d_out, p_out,
                 idx_d, idx_p, rows_d, rows_p, sem):
    wid = lax.axis_index("s") * NC + lax.axis_index("c")
    rbase = wid * CHUNKS_PER_W
    pltpu.sync_copy(d_ids.at[pl.ds(rbase, CHUNKS_PER_W)], idx_d)
    pltpu.sync_copy(p_ids.at[pl.ds(rbase, CHUNKS_PER_W)], idx_p)
    _remap(idx_d)
    _remap(idx_p)
    copies = []
    for j in range(CHUNKS_PER_W):
        copies.append(pltpu.async_copy(
            d_lin.at[idx_d.at[j]], rows_d.at[pl.ds(j * CHUNK, CHUNK)], sem))
        copies.append(pltpu.async_copy(
            p_lin.at[idx_p.at[j]], rows_p.at[pl.ds(j * CHUNK, CHUNK)], sem))
    for c in copies:
        c.wait()
    half = wid // (NW // 2)
    base = (wid % (NW // 2)) * B_PER_W
    pltpu.sync_copy(rows_d,
                    d_out.at[pl.ds(base, B_PER_W), pl.ds(half * EMB, EMB)])
    pltpu.sync_copy(rows_p,
                    p_out.at[pl.ds(base, B_PER_W), pl.ds(half * EMB, EMB)])


_gather = pl.kernel(
    _gather_body,
    out_type=(
        jax.ShapeDtypeStruct((BATCH // 2, 2 * EMB), jnp.float32),
        jax.ShapeDtypeStruct((BATCH // 2, 2 * EMB), jnp.float32),
    ),
    mesh=plsc.VectorSubcoreMesh(
        core_axis_name="c", subcore_axis_name="s",
        num_cores=NC, num_subcores=NS),
    scratch_types=[
        pltpu.VMEM((CHUNKS_PER_W, CHUNK), jnp.int32),
        pltpu.VMEM((CHUNKS_PER_W, CHUNK), jnp.int32),
        pltpu.VMEM((B_PER_W, EMB), jnp.float32),
        pltpu.VMEM((B_PER_W, EMB), jnp.float32),
        pltpu.SemaphoreType.DMA,
    ],
    compiler_params=pltpu.CompilerParams(use_tc_tiling_on_sc=False),
)


def _mlp_body(d_ref, p_ref, w1a_ref, w1b_ref, b1_ref, w2_ref, b2_ref,
              otop_ref, obot_ref):
    h = jnp.dot(d_ref[...], w1a_ref[...], preferred_element_type=jnp.float32)
    h = h + jnp.dot(p_ref[...], w1b_ref[...], preferred_element_type=jnp.float32)
    h = jnp.maximum(h + b1_ref[...], 0.0)
    o = jnp.dot(h, w2_ref[...], preferred_element_type=jnp.float32) + b2_ref[...]
    o = 1.0 / (1.0 + jnp.exp(-o))
    otop_ref[...] = o[:, 0:4]
    obot_ref[...] = o[:, 4:8]


BB = 4096


def _mlp(d2, p2, w1a_bd, w1b_bd, b1_bd, w2_bd, b2_bd):
    return pl.pallas_call(
        _mlp_body,
        grid=(BATCH // 2 // BB,),
        in_specs=[
            pl.BlockSpec((BB, 128), lambda i: (i, 0)),
            pl.BlockSpec((BB, 128), lambda i: (i, 0)),
            pl.BlockSpec((128, 2 * 32), lambda i: (0, 0)),
            pl.BlockSpec((128, 2 * 32), lambda i: (0, 0)),
            pl.BlockSpec((1, 2 * 32), lambda i: (0, 0)),
            pl.BlockSpec((2 * 32, 8), lambda i: (0, 0)),
            pl.BlockSpec((1, 8), lambda i: (0, 0)),
        ],
        out_specs=[pl.BlockSpec((BB, 4), lambda i: (i, 0)),
                   pl.BlockSpec((BB, 4), lambda i: (i, 0))],
        out_shape=[jax.ShapeDtypeStruct((BATCH // 2, 4), jnp.float32),
                   jax.ShapeDtypeStruct((BATCH // 2, 4), jnp.float32)],
    )(d2, p2, w1a_bd, w1b_bd, b1_bd, w2_bd, b2_bd)


def _block_diag(w):
    r, c = w.shape
    z = jnp.zeros((r, c), dtype=w.dtype)
    return jnp.concatenate(
        [jnp.concatenate([w, z], axis=1),
         jnp.concatenate([z, w], axis=1)], axis=0)


@jax.jit
def kernel(depth_ids, purpose_ids, depth_table, purpose_table, W1, b1, W2, b2):
    d_ids = depth_ids.astype(jnp.int32).reshape(NW * CHUNKS_PER_W, CHUNK)
    p_ids = purpose_ids.astype(jnp.int32).reshape(NW * CHUNKS_PER_W, CHUNK)
    eye = jnp.eye(EMB, dtype=jnp.float32)
    lin_d, lin_p = _pack(depth_table.T, purpose_table.T, eye)
    d2, p2 = _gather(lin_d.reshape(VP, EMB), lin_p.reshape(VP, EMB),
                     d_ids, p_ids)
    w1a_bd = _block_diag(W1[:EMB])
    w1b_bd = _block_diag(W1[EMB:])
    b1_bd = jnp.concatenate([b1, b1]).reshape(1, 2 * 32)
    w2_bd = _block_diag(W2)
    b2_bd = jnp.concatenate([b2, b2]).reshape(1, 8)
    o_top, o_bot = _mlp(d2, p2, w1a_bd, w1b_bd, b1_bd, w2_bd, b2_bd)
    return jnp.concatenate([o_top, o_bot], axis=0)
